# Initial kernel scaffold; baseline (speedup 1.0000x reference)
import jax
import jax.numpy as jnp
from jax.experimental import pallas as pl


def kernel(x, edge_index, W1, b1, W2, b2, W3, b3, Wc1, bc1, Wc2, bc2):
    raise NotImplementedError("write your pallas kernel here")


# speedup vs baseline: 11.4835x; 11.4835x over previous
import jax
import jax.numpy as jnp
from jax import lax
from jax.experimental import pallas as pl
from jax.experimental.pallas import tpu as pltpu
from jax.experimental.pallas import tpu_sc as plsc

NC = 2
NS = 16
CW = 128
BR = 512
F32 = jnp.float32


def _sc_mesh():
    return plsc.VectorSubcoreMesh(core_axis_name="c", subcore_axis_name="s",
                                  num_cores=NC, num_subcores=NS)


def _build_deg(n_pad, ch):
    rpt = n_pad // NS

    def body(dsts, out, dst_v, buf, acc):
        c = lax.axis_index("c")
        s = lax.axis_index("s")
        wid = c * NS + s
        base = s * rpt

        def fill(val):
            def f(r, _):
                buf[r] = jnp.full((16,), val, F32)
                return 0
            lax.fori_loop(0, CW, f, 0)

        fill(0.0)
        for k in range(rpt // CW):
            pltpu.sync_copy(buf, acc.at[pl.ds(base + k * CW, CW)])
        fill(1.0)
        pltpu.sync_copy(dsts.at[wid], dst_v)
        plsc.subcore_barrier()

        def step(j, carry):
            pltpu.sync_copy(buf, acc.at[dst_v.at[j]], add=True)
            return carry
        lax.fori_loop(0, ch, step, 0)
        plsc.subcore_barrier()
        pltpu.sync_copy(acc.at[pl.ds(base, rpt)], out.at[wid])

    return pl.kernel(
        body,
        out_type=jax.ShapeDtypeStruct((NC * NS, rpt, 16), F32),
        mesh=_sc_mesh(),
        scratch_types=[
            pltpu.VMEM((ch, CW), jnp.int32),
            pltpu.VMEM((CW, 16), F32),
            pltpu.VMEM_SHARED((n_pad, 16), F32),
        ],
    )


def _build_prop(n_pad, ch, h):
    rpt = n_pad // NS

    def body(g, srcs, dsts, out, src_v, dst_v, rows, acc, sem):
        c = lax.axis_index("c")
        s = lax.axis_index("s")
        wid = c * NS + s
        base = s * rpt
        vpr = h // 16

        def z(i, carry):
            rows[i // vpr, pl.ds((i % vpr) * 16, 16)] = jnp.zeros((16,), F32)
            return carry
        lax.fori_loop(0, CW * vpr, z, 0)
        for k in range(rpt // CW):
            pltpu.sync_copy(rows, acc.at[pl.ds(base + k * CW, CW)])
        pltpu.sync_copy(srcs.at[wid], src_v)
        pltpu.sync_copy(dsts.at[wid], dst_v)
        plsc.subcore_barrier()

        def step(j, carry):
            pltpu.async_copy(g.at[src_v.at[j]], rows, sem).wait()
            pltpu.sync_copy(rows, acc.at[dst_v.at[j]], add=True)
            return carry
        lax.fori_loop(0, ch, step, 0)
        plsc.subcore_barrier()
        pltpu.sync_copy(acc.at[pl.ds(base, rpt)], out.at[wid])

    return pl.kernel(
        body,
        out_type=jax.ShapeDtypeStruct((NC * NS, rpt, h), F32),
        mesh=_sc_mesh(),
        scratch_types=[
            pltpu.VMEM((ch, CW), jnp.int32),
            pltpu.VMEM((ch, CW), jnp.int32),
            pltpu.VMEM((CW, h), F32),
            pltpu.VMEM_SHARED((n_pad, h), F32),
            pltpu.SemaphoreType.DMA,
        ],
    )


def _tc_first(xp, W, deg16):
    n_pad, d = xp.shape
    hd = W.shape[1]
    nb = n_pad // BR

    def body(x_ref, w_ref, dg_ref, g_ref, db_ref):
        deg = dg_ref[0, :, 0:1] + dg_ref[1, :, 0:1] + 1.0
        db = jnp.broadcast_to(lax.rsqrt(deg), (BR, hd))
        db_ref[...] = db
        g_ref[...] = db * jnp.dot(x_ref[...], w_ref[...],
                                  preferred_element_type=F32)

    return pl.pallas_call(
        body,
        grid=(nb,),
        in_specs=[
            pl.BlockSpec((BR, d), lambda i: (i, 0)),
            pl.BlockSpec((d, hd), lambda i: (0, 0)),
            pl.BlockSpec((2, BR, 16), lambda i: (0, i, 0)),
        ],
        out_specs=[
            pl.BlockSpec((BR, hd), lambda i: (i, 0)),
            pl.BlockSpec((BR, hd), lambda i: (i, 0)),
        ],
        out_shape=[jax.ShapeDtypeStruct((n_pad, hd), F32),
                   jax.ShapeDtypeStruct((n_pad, hd), F32)],
    )(xp, W, deg16)


def _tc_mid(s2, g, db, b, W):
    n_pad, hd = g.shape
    nb = n_pad // BR

    def body(s_ref, g_ref, db_ref, b_ref, w_ref, o_ref):
        dv = db_ref[...]
        hcur = jnp.maximum(dv * (s_ref[0] + s_ref[1] + g_ref[...]) + b_ref[...], 0.0)
        o_ref[...] = dv * jnp.dot(hcur, w_ref[...], preferred_element_type=F32)

    return pl.pallas_call(
        body,
        grid=(nb,),
        in_specs=[
            pl.BlockSpec((2, BR, hd), lambda i: (0, i, 0)),
            pl.BlockSpec((BR, hd), lambda i: (i, 0)),
            pl.BlockSpec((BR, hd), lambda i: (i, 0)),
            pl.BlockSpec((1, hd), lambda i: (0, 0)),
            pl.BlockSpec((hd, hd), lambda i: (0, 0)),
        ],
        out_specs=pl.BlockSpec((BR, hd), lambda i: (i, 0)),
        out_shape=jax.ShapeDtypeStruct((n_pad, hd), F32),
    )(s2, g, db, b, W)


def _tc_final(s2, g, db, b, wc1, bc1, wc2, bc2, n):
    n_pad, hd = g.shape
    nb = n_pad // BR

    def body(s_ref, g_ref, db_ref, b_ref, w1_ref, b1_ref, w2_ref, b2_ref,
             o_ref, acc):
        i = pl.program_id(0)
        dv = db_ref[...]
        hcur = jnp.maximum(dv * (s_ref[0] + s_ref[1] + g_ref[...]) + b_ref[...], 0.0)
        row = lax.broadcasted_iota(jnp.int32, (BR, 1), 0) + i * BR
        hcur = jnp.where(row < n, hcur, 0.0)
        part = jnp.sum(hcur, axis=0, keepdims=True)

        @pl.when(i == 0)
        def _():
            acc[...] = jnp.zeros_like(acc)

        acc[0:1, :] = acc[0:1, :] + part

        @pl.when(i == nb - 1)
        def _():
            pooled = acc[0:1, :] * (1.0 / n)
            cc = jnp.maximum(jnp.dot(pooled, w1_ref[...],
                                     preferred_element_type=F32) + b1_ref[...], 0.0)
            oo = jnp.dot(cc, w2_ref[...], preferred_element_type=F32) + b2_ref[...]
            o_ref[...] = jax.nn.sigmoid(oo)

    return pl.pallas_call(
        body,
        grid=(nb,),
        in_specs=[
            pl.BlockSpec((2, BR, hd), lambda i: (0, i, 0)),
            pl.BlockSpec((BR, hd), lambda i: (i, 0)),
            pl.BlockSpec((BR, hd), lambda i: (i, 0)),
            pl.BlockSpec((1, hd), lambda i: (0, 0)),
            pl.BlockSpec((hd, 128), lambda i: (0, 0)),
            pl.BlockSpec((1, 128), lambda i: (0, 0)),
            pl.BlockSpec((128, 128), lambda i: (0, 0)),
            pl.BlockSpec((1, 128), lambda i: (0, 0)),
        ],
        out_specs=pl.BlockSpec((1, 128), lambda i: (0, 0)),
        out_shape=jax.ShapeDtypeStruct((1, 128), F32),
        scratch_shapes=[pltpu.VMEM((8, 128), F32)],
    )(s2, g, db, b, wc1, bc1, wc2, bc2)


def kernel(x, edge_index, W1, b1, W2, b2, W3, b3, Wc1, bc1, Wc2, bc2):
    n, d = x.shape
    hd = W1.shape[1]
    e = edge_index.shape[1]
    n_pad = -(-n // (NS * CW)) * (NS * CW)
    ch = -(-e // (NC * NS * CW))
    e_pad = NC * NS * CW * ch

    xp = jnp.zeros((n_pad, d), F32).at[:n, :].set(x)
    pad = jnp.full((e_pad - e,), n_pad - 1, jnp.int32)
    srcp = jnp.concatenate([edge_index[0].astype(jnp.int32), pad]).reshape(NC * NS, ch, CW)
    dstp = jnp.concatenate([edge_index[1].astype(jnp.int32), pad]).reshape(NC * NS, ch, CW)

    deg16 = _build_deg(n_pad, ch)(dstp).reshape(NC, n_pad, 16)
    g1, db = _tc_first(xp, W1, deg16)
    prop = _build_prop(n_pad, ch, hd)

    s = prop(g1, srcp, dstp).reshape(NC, n_pad, hd)
    g2 = _tc_mid(s, g1, db, b1.reshape(1, hd), W2)
    s = prop(g2, srcp, dstp).reshape(NC, n_pad, hd)
    g3 = _tc_mid(s, g2, db, b2.reshape(1, hd), W3)
    s = prop(g3, srcp, dstp).reshape(NC, n_pad, hd)

    hh = Wc1.shape[1]
    wc1p = jnp.zeros((hd, 128), F32).at[:, :hh].set(Wc1)
    bc1p = jnp.zeros((1, 128), F32).at[0, :hh].set(bc1)
    wc2p = jnp.zeros((128, 128), F32).at[:hh, 0].set(Wc2[:, 0])
    bc2p = jnp.zeros((1, 128), F32).at[0, 0].set(bc2[0])
    o = _tc_final(s, g3, db, b3.reshape(1, hd), wc1p, bc1p, wc2p, bc2p, n)
    return o[:1, :1]
